# baseline (speedup 1.0000x reference)
import functools

import jax
import jax.numpy as jnp
from jax import lax
from jax.experimental import pallas as pl
from jax.experimental.pallas import tpu as pltpu
from jax.experimental.pallas import tpu_sc as plsc

NUM_SPECIES = 100000
D_MODEL = 64
BATCH = 16384

_info = plsc.get_sparse_core_info()
_NC, _NS = _info.num_cores, _info.num_subcores
_NW = _NC * _NS
_CHUNK = 128
_B_PER_W = BATCH // _NW
_NCHUNK = _B_PER_W // _CHUNK

_SB = 128
_NFULL = NUM_SPECIES // _SB
_TAIL = NUM_SPECIES - _NFULL * _SB
_KMAX = (_NFULL + _NW - 1) // _NW

_mesh = plsc.VectorSubcoreMesh(core_axis_name="c", subcore_axis_name="s")
_params = pltpu.CompilerParams(use_tc_tiling_on_sc=True,
                               needs_layout_passes=False,
                               disable_bounds_checks=True)


_BLKW = D_MODEL * _SB


@functools.partial(
    pl.kernel,
    mesh=_mesh,
    compiler_params=_params,
    out_type=jax.ShapeDtypeStruct((NUM_SPECIES // 2, 2 * D_MODEL), jnp.float32),
    scratch_types=[
        pltpu.VMEM((2, D_MODEL, _SB), jnp.float32),
        pltpu.VMEM((2, _SB // 2, 2 * D_MODEL), jnp.float32),
        pltpu.SemaphoreType.DMA((2,)),
        pltpu.SemaphoreType.DMA((2,)),
    ],
)
def _relayout_kernel(tt_hbm, tail_hbm, out_hbm, inb, outb, sem_i, sem_o):
    wid = lax.axis_index("s") * _NC + lax.axis_index("c")
    nblk = (_NFULL - wid + _NW - 1) // _NW
    iota16 = lax.broadcasted_iota(jnp.int32, (16,), 0)

    def start_in(k, p):
        col = pl.multiple_of((wid + _NW * k) * _SB, _SB)
        pltpu.async_copy(
            tt_hbm.at[:, pl.ds(col, _SB)], inb.at[p], sem_i.at[p])

    def start_out(k, p):
        row = pl.multiple_of((wid + _NW * k) * (_SB // 2), _SB // 2)
        pltpu.async_copy(
            outb.at[p], out_hbm.at[pl.ds(row, _SB // 2)], sem_o.at[p])

    def wait_in(p):
        pltpu.make_async_copy(tt_hbm.at[:, pl.ds(0, _SB)], inb.at[p],
                              sem_i.at[p]).wait()

    def wait_out(p):
        pltpu.make_async_copy(outb.at[p], out_hbm.at[pl.ds(0, _SB // 2)],
                              sem_o.at[p]).wait()

    def transpose_block(p):
        @plsc.parallel_loop(0, _SB // 16, unroll=2)
        def body(s0):
            cols = s0 * 16 + iota16
            rows = s0 * 8 + (iota16 >> 1)
            half = (iota16 & 1) * D_MODEL
            for j in range(16):
                perm = (iota16 + j) & 15
                for c0 in range(4):
                    vals = plsc.load_gather(inb.at[p], [c0 * 16 + perm, cols])
                    plsc.store_scatter(outb.at[p],
                                       [rows, half + c0 * 16 + perm], vals)

    start_in(0, 0)

    def blk_body(k, carry):
        p = k & 1

        @pl.when(k + 1 < nblk)
        def _():
            start_in(k + 1, 1 - p)

        wait_in(p)

        @pl.when(k >= 2)
        def _():
            wait_out(p)

        transpose_block(p)
        start_out(k, p)
        return carry

    lax.fori_loop(0, nblk, blk_body, 0)
    wait_out((nblk - 1) & 1)
    wait_out((nblk - 2) & 1)

    @pl.when(wid == 0)
    def _():
        pltpu.sync_copy(tail_hbm, outb.at[0, pl.ds(0, _TAIL // 2)])
        pltpu.sync_copy(outb.at[0, pl.ds(0, _TAIL // 2)],
                        out_hbm.at[pl.ds(_NFULL * (_SB // 2), _TAIL // 2)])


@functools.partial(
    pl.kernel,
    mesh=_mesh,
    compiler_params=_params,
    out_type=jax.ShapeDtypeStruct((BATCH, 2 * D_MODEL), jnp.float32),
    scratch_types=[
        pltpu.VMEM((_B_PER_W,), jnp.int32),
        pltpu.VMEM((_B_PER_W, 2 * D_MODEL), jnp.float32),
        pltpu.SemaphoreType.DMA,
    ],
)
def _gather_kernel(idx_hbm, table_hbm, out_hbm, idx_v, rows_v, sem):
    wid = lax.axis_index("s") * _NC + lax.axis_index("c")
    base = wid * _B_PER_W
    pltpu.sync_copy(idx_hbm.at[pl.ds(base, _B_PER_W)], idx_v)
    copies = [
        pltpu.async_copy(
            table_hbm.at[idx_v.at[pl.ds(j * _CHUNK, _CHUNK)]],
            rows_v.at[pl.ds(j * _CHUNK, _CHUNK)],
            sem,
        )
        for j in range(_NCHUNK)
    ]
    for c in copies:
        c.wait()
    pltpu.sync_copy(rows_v, out_hbm.at[pl.ds(base, _B_PER_W)])


def kernel(species_ids, embedding):
    idx = species_ids.astype(jnp.int32)
    tt = embedding.T
    tail = embedding[_NFULL * _SB:].reshape(_TAIL // 2, 2 * D_MODEL)
    table2 = _relayout_kernel(tt, tail)
    wide = _gather_kernel(idx >> 1, table2)
    pairs = wide.reshape(BATCH, 2, D_MODEL)
    out = jnp.where((idx & 1)[:, None] == 1, pairs[:, 1, :], pairs[:, 0, :])
    return out[:, None, :]


# speedup vs baseline: 1.4613x; 1.4613x over previous
import functools

import jax
import jax.numpy as jnp
from jax import lax
from jax.experimental import pallas as pl
from jax.experimental.pallas import tpu as pltpu
from jax.experimental.pallas import tpu_sc as plsc

NUM_SPECIES = 100000
D_MODEL = 64
BATCH = 16384

_info = plsc.get_sparse_core_info()
_NC, _NS = _info.num_cores, _info.num_subcores
_NW = _NC * _NS
_CHUNK = 128
_B_PER_W = BATCH // _NW
_NCHUNK = _B_PER_W // _CHUNK

_SB = 128
_NFULL = NUM_SPECIES // _SB
_TAIL = NUM_SPECIES - _NFULL * _SB
_KMAX = (_NFULL + _NW - 1) // _NW

_mesh = plsc.VectorSubcoreMesh(core_axis_name="c", subcore_axis_name="s")
_params = pltpu.CompilerParams(use_tc_tiling_on_sc=True,
                               needs_layout_passes=False,
                               disable_bounds_checks=True)


_BLKW = D_MODEL * _SB


@functools.partial(
    pl.kernel,
    mesh=_mesh,
    compiler_params=_params,
    out_type=jax.ShapeDtypeStruct((NUM_SPECIES // 2, 2 * D_MODEL), jnp.float32),
    scratch_types=[
        pltpu.VMEM((2, D_MODEL, _SB), jnp.float32),
        pltpu.VMEM((2, _SB // 2, 2 * D_MODEL), jnp.float32),
        pltpu.SemaphoreType.DMA((2,)),
        pltpu.SemaphoreType.DMA((2,)),
    ],
)
def _relayout_kernel(tt_hbm, tail_hbm, out_hbm, inb, outb, sem_i, sem_o):
    wid = lax.axis_index("s") * _NC + lax.axis_index("c")
    nblk = (_NFULL - wid + _NW - 1) // _NW
    iota16 = lax.broadcasted_iota(jnp.int32, (16,), 0)

    def start_in(k, p):
        col = pl.multiple_of((wid + _NW * k) * _SB, _SB)
        pltpu.async_copy(
            tt_hbm.at[:, pl.ds(col, _SB)], inb.at[p], sem_i.at[p])

    def start_out(k, p):
        row = pl.multiple_of((wid + _NW * k) * (_SB // 2), _SB // 2)
        pltpu.async_copy(
            outb.at[p], out_hbm.at[pl.ds(row, _SB // 2)], sem_o.at[p])

    def wait_in(p):
        pltpu.make_async_copy(tt_hbm.at[:, pl.ds(0, _SB)], inb.at[p],
                              sem_i.at[p]).wait()

    def wait_out(p):
        pltpu.make_async_copy(outb.at[p], out_hbm.at[pl.ds(0, _SB // 2)],
                              sem_o.at[p]).wait()

    def transpose_block(p):
        @plsc.parallel_loop(0, _SB // 16, unroll=4)
        def body(s0):
            cols = s0 * 16 + iota16
            rows = s0 * 8 + (iota16 >> 1)
            half = (iota16 & 1) * D_MODEL
            for j in range(16):
                perm = (iota16 + j) & 15
                for c0 in range(4):
                    vals = plsc.load_gather(inb.at[p], [c0 * 16 + perm, cols])
                    plsc.store_scatter(outb.at[p],
                                       [rows, half + c0 * 16 + perm], vals)

    start_in(0, 0)

    def blk_body(k, carry):
        p = k & 1

        @pl.when(k + 1 < nblk)
        def _():
            start_in(k + 1, 1 - p)

        wait_in(p)

        @pl.when(k >= 2)
        def _():
            wait_out(p)

        transpose_block(p)
        start_out(k, p)
        return carry

    lax.fori_loop(0, nblk, blk_body, 0)
    wait_out((nblk - 1) & 1)
    wait_out((nblk - 2) & 1)

    @pl.when(wid == 0)
    def _():
        pltpu.sync_copy(tail_hbm, outb.at[0, pl.ds(0, _TAIL // 2)])
        pltpu.sync_copy(outb.at[0, pl.ds(0, _TAIL // 2)],
                        out_hbm.at[pl.ds(_NFULL * (_SB // 2), _TAIL // 2)])


@functools.partial(
    pl.kernel,
    mesh=_mesh,
    compiler_params=_params,
    out_type=jax.ShapeDtypeStruct((BATCH, 2 * D_MODEL), jnp.float32),
    scratch_types=[
        pltpu.VMEM((_B_PER_W,), jnp.int32),
        pltpu.VMEM((_B_PER_W, 2 * D_MODEL), jnp.float32),
        pltpu.SemaphoreType.DMA,
    ],
)
def _gather_kernel(idx_hbm, table_hbm, out_hbm, idx_v, rows_v, sem):
    wid = lax.axis_index("s") * _NC + lax.axis_index("c")
    base = wid * _B_PER_W
    pltpu.sync_copy(idx_hbm.at[pl.ds(base, _B_PER_W)], idx_v)
    copies = [
        pltpu.async_copy(
            table_hbm.at[idx_v.at[pl.ds(j * _CHUNK, _CHUNK)]],
            rows_v.at[pl.ds(j * _CHUNK, _CHUNK)],
            sem,
        )
        for j in range(_NCHUNK)
    ]
    for c in copies:
        c.wait()
    pltpu.sync_copy(rows_v, out_hbm.at[pl.ds(base, _B_PER_W)])


def kernel(species_ids, embedding):
    idx = species_ids.astype(jnp.int32)
    tt = embedding.T
    tail = embedding[_NFULL * _SB:].reshape(_TAIL // 2, 2 * D_MODEL)
    table2 = _relayout_kernel(tt, tail)
    wide = _gather_kernel(idx >> 1, table2)
    pairs = wide.reshape(BATCH, 2, D_MODEL)
    out = jnp.where((idx & 1)[:, None] == 1, pairs[:, 1, :], pairs[:, 0, :])
    return out[:, None, :]


# speedup vs baseline: 1.6529x; 1.1311x over previous
import functools

import jax
import jax.numpy as jnp
from jax import lax
from jax.experimental import pallas as pl
from jax.experimental.pallas import tpu as pltpu
from jax.experimental.pallas import tpu_sc as plsc

NUM_SPECIES = 100000
D_MODEL = 64
BATCH = 16384

_info = plsc.get_sparse_core_info()
_NC, _NS = _info.num_cores, _info.num_subcores
_NW = _NC * _NS
_CHUNK = 128
_B_PER_W = BATCH // _NW
_NCHUNK = _B_PER_W // _CHUNK

_SB = 128
_NFULL = NUM_SPECIES // _SB
_TAIL = NUM_SPECIES - _NFULL * _SB
_KMAX = (_NFULL + _NW - 1) // _NW

_mesh = plsc.VectorSubcoreMesh(core_axis_name="c", subcore_axis_name="s")
_params = pltpu.CompilerParams(use_tc_tiling_on_sc=True,
                               needs_layout_passes=False,
                               disable_bounds_checks=True)


_BLKW = D_MODEL * _SB


@functools.partial(
    pl.kernel,
    mesh=_mesh,
    compiler_params=_params,
    out_type=jax.ShapeDtypeStruct((NUM_SPECIES // 2, 2 * D_MODEL), jnp.float32),
    scratch_types=[
        pltpu.VMEM((2, D_MODEL, _SB), jnp.float32),
        pltpu.VMEM((2, _SB // 2, 2 * D_MODEL), jnp.float32),
        pltpu.SemaphoreType.DMA((2,)),
        pltpu.SemaphoreType.DMA((2,)),
    ],
)
def _relayout_kernel(tt_hbm, tail_hbm, out_hbm, inb, outb, sem_i, sem_o):
    wid = lax.axis_index("s") * _NC + lax.axis_index("c")
    nblk = (_NFULL - wid + _NW - 1) // _NW
    iota16 = lax.broadcasted_iota(jnp.int32, (16,), 0)

    def start_in(k, p):
        col = pl.multiple_of((wid + _NW * k) * _SB, _SB)
        pltpu.async_copy(
            tt_hbm.at[:, pl.ds(col, _SB)], inb.at[p], sem_i.at[p])

    def start_out(k, p):
        row = pl.multiple_of((wid + _NW * k) * (_SB // 2), _SB // 2)
        pltpu.async_copy(
            outb.at[p], out_hbm.at[pl.ds(row, _SB // 2)], sem_o.at[p])

    def wait_in(p):
        pltpu.make_async_copy(tt_hbm.at[:, pl.ds(0, _SB)], inb.at[p],
                              sem_i.at[p]).wait()

    def wait_out(p):
        pltpu.make_async_copy(outb.at[p], out_hbm.at[pl.ds(0, _SB // 2)],
                              sem_o.at[p]).wait()

    def transpose_block(p):
        @plsc.parallel_loop(0, _SB // 16, unroll=4)
        def body(s0):
            cols = s0 * 16 + iota16
            rows = s0 * 8 + (iota16 >> 1)
            half = (iota16 & 1) * D_MODEL
            for j in range(16):
                perm = (iota16 + j) & 15
                for c0 in range(4):
                    vals = plsc.load_gather(inb.at[p], [c0 * 16 + perm, cols])
                    plsc.store_scatter(outb.at[p],
                                       [rows, half + c0 * 16 + perm], vals)

    start_in(0, 0)

    def blk_body(k, carry):
        p = k & 1

        @pl.when(k + 1 < nblk)
        def _():
            start_in(k + 1, 1 - p)

        wait_in(p)

        @pl.when(k >= 2)
        def _():
            wait_out(p)

        transpose_block(p)
        start_out(k, p)
        return carry

    lax.fori_loop(0, nblk, blk_body, 0)
    wait_out((nblk - 1) & 1)
    wait_out((nblk - 2) & 1)

    @pl.when(wid == 0)
    def _():
        pltpu.sync_copy(tail_hbm, outb.at[0, pl.ds(0, _TAIL // 2)])
        pltpu.sync_copy(outb.at[0, pl.ds(0, _TAIL // 2)],
                        out_hbm.at[pl.ds(_NFULL * (_SB // 2), _TAIL // 2)])


@functools.partial(
    pl.kernel,
    mesh=_mesh,
    compiler_params=_params,
    out_type=jax.ShapeDtypeStruct((D_MODEL, BATCH), jnp.float32),
    scratch_types=[
        pltpu.VMEM((_B_PER_W,), jnp.int32),
        pltpu.VMEM((_B_PER_W,), jnp.int32),
        pltpu.VMEM((_B_PER_W,), jnp.int32),
        pltpu.VMEM((_B_PER_W, 2 * D_MODEL), jnp.float32),
        pltpu.VMEM((D_MODEL, _B_PER_W), jnp.float32),
        pltpu.SemaphoreType.DMA,
    ],
)
def _gather_kernel(idx_hbm, table_hbm, out_hbm, idx_v, pair_v, half_v,
                   rows_v, obuf, sem):
    wid = lax.axis_index("s") * _NC + lax.axis_index("c")
    base = wid * _B_PER_W
    iota16 = lax.broadcasted_iota(jnp.int32, (16,), 0)
    pltpu.sync_copy(idx_hbm.at[pl.ds(base, _B_PER_W)], idx_v)

    @plsc.parallel_loop(0, _B_PER_W // 16, unroll=4)
    def _prep(g):
        v = idx_v[pl.ds(g * 16, 16)]
        pair_v[pl.ds(g * 16, 16)] = v >> 1
        half_v[pl.ds(g * 16, 16)] = (v & 1) * D_MODEL

    copies = [
        pltpu.async_copy(
            table_hbm.at[pair_v.at[pl.ds(j * _CHUNK, _CHUNK)]],
            rows_v.at[pl.ds(j * _CHUNK, _CHUNK)],
            sem,
        )
        for j in range(_NCHUNK)
    ]
    for c in copies:
        c.wait()

    @plsc.parallel_loop(0, _B_PER_W // 16, unroll=4)
    def _tr(b0):
        halfvec = half_v[pl.ds(b0 * 16, 16)]
        brows = b0 * 16 + iota16
        for j in range(16):
            perm = (iota16 + j) & 15
            for c0 in range(4):
                vals = plsc.load_gather(
                    rows_v, [brows, halfvec + c0 * 16 + perm])
                plsc.store_scatter(obuf, [c0 * 16 + perm, brows], vals)

    pltpu.sync_copy(obuf, out_hbm.at[:, pl.ds(base, _B_PER_W)])


def kernel(species_ids, embedding):
    idx = species_ids.astype(jnp.int32)
    tt = embedding.T
    tail = embedding[_NFULL * _SB:].reshape(_TAIL // 2, 2 * D_MODEL)
    table2 = _relayout_kernel(tt, tail)
    out_cm = _gather_kernel(idx, table2)
    return out_cm.T[:, None, :]


# speedup vs baseline: 1.7294x; 1.0462x over previous
import functools

import jax
import jax.numpy as jnp
from jax import lax
from jax.experimental import pallas as pl
from jax.experimental.pallas import tpu as pltpu
from jax.experimental.pallas import tpu_sc as plsc

NUM_SPECIES = 100000
D_MODEL = 64
BATCH = 16384

_info = plsc.get_sparse_core_info()
_NC, _NS = _info.num_cores, _info.num_subcores
_NW = _NC * _NS
_CHUNK = 128
_B_PER_W = BATCH // _NW
_NCHUNK = _B_PER_W // _CHUNK

_SB = 256
_NFULL = NUM_SPECIES // _SB
_TAIL = NUM_SPECIES - _NFULL * _SB
_KMAX = (_NFULL + _NW - 1) // _NW

_mesh = plsc.VectorSubcoreMesh(core_axis_name="c", subcore_axis_name="s")
_params = pltpu.CompilerParams(use_tc_tiling_on_sc=True,
                               needs_layout_passes=False,
                               disable_bounds_checks=True)


_BLKW = D_MODEL * _SB


@functools.partial(
    pl.kernel,
    mesh=_mesh,
    compiler_params=_params,
    out_type=jax.ShapeDtypeStruct((NUM_SPECIES // 2, 2 * D_MODEL), jnp.float32),
    scratch_types=[
        pltpu.VMEM((2, D_MODEL, _SB), jnp.float32),
        pltpu.VMEM((2, _SB // 2, 2 * D_MODEL), jnp.float32),
        pltpu.SemaphoreType.DMA((2,)),
        pltpu.SemaphoreType.DMA((2,)),
    ],
)
def _relayout_kernel(tt_hbm, tail_hbm, out_hbm, inb, outb, sem_i, sem_o):
    wid = lax.axis_index("s") * _NC + lax.axis_index("c")
    nblk = (_NFULL - wid + _NW - 1) // _NW
    iota16 = lax.broadcasted_iota(jnp.int32, (16,), 0)

    def start_in(k, p):
        col = pl.multiple_of((wid + _NW * k) * _SB, _SB)
        pltpu.async_copy(
            tt_hbm.at[:, pl.ds(col, _SB)], inb.at[p], sem_i.at[p])

    def start_out(k, p):
        row = pl.multiple_of((wid + _NW * k) * (_SB // 2), _SB // 2)
        pltpu.async_copy(
            outb.at[p], out_hbm.at[pl.ds(row, _SB // 2)], sem_o.at[p])

    def wait_in(p):
        pltpu.make_async_copy(tt_hbm.at[:, pl.ds(0, _SB)], inb.at[p],
                              sem_i.at[p]).wait()

    def wait_out(p):
        pltpu.make_async_copy(outb.at[p], out_hbm.at[pl.ds(0, _SB // 2)],
                              sem_o.at[p]).wait()

    def transpose_block(p):
        @plsc.parallel_loop(0, _SB // 16, unroll=4)
        def body(s0):
            cols = s0 * 16 + iota16
            rows = s0 * 8 + (iota16 >> 1)
            half = (iota16 & 1) * D_MODEL
            for j in range(16):
                perm = (iota16 + j) & 15
                for c0 in range(4):
                    vals = plsc.load_gather(inb.at[p], [c0 * 16 + perm, cols])
                    plsc.store_scatter(outb.at[p],
                                       [rows, half + c0 * 16 + perm], vals)

    start_in(0, 0)

    def blk_body(k, carry):
        p = k & 1

        @pl.when(k + 1 < nblk)
        def _():
            start_in(k + 1, 1 - p)

        wait_in(p)

        @pl.when(k >= 2)
        def _():
            wait_out(p)

        transpose_block(p)
        start_out(k, p)
        return carry

    lax.fori_loop(0, nblk, blk_body, 0)
    wait_out((nblk - 1) & 1)
    wait_out((nblk - 2) & 1)

    @pl.when(wid == 0)
    def _():
        pltpu.sync_copy(tail_hbm, outb.at[0, pl.ds(0, _TAIL // 2)])
        pltpu.sync_copy(outb.at[0, pl.ds(0, _TAIL // 2)],
                        out_hbm.at[pl.ds(_NFULL * (_SB // 2), _TAIL // 2)])


@functools.partial(
    pl.kernel,
    mesh=_mesh,
    compiler_params=_params,
    out_type=jax.ShapeDtypeStruct((D_MODEL, BATCH), jnp.float32),
    scratch_types=[
        pltpu.VMEM((_B_PER_W,), jnp.int32),
        pltpu.VMEM((_B_PER_W,), jnp.int32),
        pltpu.VMEM((_B_PER_W,), jnp.int32),
        pltpu.VMEM((_B_PER_W, 2 * D_MODEL), jnp.float32),
        pltpu.VMEM((D_MODEL, _B_PER_W), jnp.float32),
        pltpu.SemaphoreType.DMA,
    ],
)
def _gather_kernel(idx_hbm, table_hbm, out_hbm, idx_v, pair_v, half_v,
                   rows_v, obuf, sem):
    wid = lax.axis_index("s") * _NC + lax.axis_index("c")
    base = wid * _B_PER_W
    iota16 = lax.broadcasted_iota(jnp.int32, (16,), 0)
    pltpu.sync_copy(idx_hbm.at[pl.ds(base, _B_PER_W)], idx_v)

    @plsc.parallel_loop(0, _B_PER_W // 16, unroll=4)
    def _prep(g):
        v = idx_v[pl.ds(g * 16, 16)]
        pair_v[pl.ds(g * 16, 16)] = v >> 1
        half_v[pl.ds(g * 16, 16)] = (v & 1) * D_MODEL

    copies = [
        pltpu.async_copy(
            table_hbm.at[pair_v.at[pl.ds(j * _CHUNK, _CHUNK)]],
            rows_v.at[pl.ds(j * _CHUNK, _CHUNK)],
            sem,
        )
        for j in range(_NCHUNK)
    ]
    for c in copies:
        c.wait()

    @plsc.parallel_loop(0, _B_PER_W // 16, unroll=4)
    def _tr(b0):
        halfvec = half_v[pl.ds(b0 * 16, 16)]
        brows = b0 * 16 + iota16
        for j in range(16):
            perm = (iota16 + j) & 15
            for c0 in range(4):
                vals = plsc.load_gather(
                    rows_v, [brows, halfvec + c0 * 16 + perm])
                plsc.store_scatter(obuf, [c0 * 16 + perm, brows], vals)

    pltpu.sync_copy(obuf, out_hbm.at[:, pl.ds(base, _B_PER_W)])


def kernel(species_ids, embedding):
    idx = species_ids.astype(jnp.int32)
    tt = embedding.T
    tail = embedding[_NFULL * _SB:].reshape(_TAIL // 2, 2 * D_MODEL)
    table2 = _relayout_kernel(tt, tail)
    out_cm = _gather_kernel(idx, table2)
    return out_cm.T[:, None, :]


# speedup vs baseline: 1.7543x; 1.0144x over previous
import functools

import jax
import jax.numpy as jnp
from jax import lax
from jax.experimental import pallas as pl
from jax.experimental.pallas import tpu as pltpu
from jax.experimental.pallas import tpu_sc as plsc

NUM_SPECIES = 100000
D_MODEL = 64
BATCH = 16384

_info = plsc.get_sparse_core_info()
_NC, _NS = _info.num_cores, _info.num_subcores
_NW = _NC * _NS
_CHUNK = 128
_B_PER_W = BATCH // _NW
_NCHUNK = _B_PER_W // _CHUNK

_SB = 384
_NFULL = NUM_SPECIES // _SB
_TAIL = NUM_SPECIES - _NFULL * _SB
_KMAX = (_NFULL + _NW - 1) // _NW

_mesh = plsc.VectorSubcoreMesh(core_axis_name="c", subcore_axis_name="s")
_params = pltpu.CompilerParams(use_tc_tiling_on_sc=True,
                               needs_layout_passes=False,
                               disable_bounds_checks=True)


_BLKW = D_MODEL * _SB


@functools.partial(
    pl.kernel,
    mesh=_mesh,
    compiler_params=_params,
    out_type=jax.ShapeDtypeStruct((NUM_SPECIES // 2, 2 * D_MODEL), jnp.float32),
    scratch_types=[
        pltpu.VMEM((2, D_MODEL, _SB), jnp.float32),
        pltpu.VMEM((2, _SB // 2, 2 * D_MODEL), jnp.float32),
        pltpu.SemaphoreType.DMA((2,)),
        pltpu.SemaphoreType.DMA((2,)),
    ],
)
def _relayout_kernel(tt_hbm, tail_hbm, out_hbm, inb, outb, sem_i, sem_o):
    wid = lax.axis_index("s") * _NC + lax.axis_index("c")
    nblk = (_NFULL - wid + _NW - 1) // _NW
    iota16 = lax.broadcasted_iota(jnp.int32, (16,), 0)

    def start_in(k, p):
        col = pl.multiple_of((wid + _NW * k) * _SB, _SB)
        pltpu.async_copy(
            tt_hbm.at[:, pl.ds(col, _SB)], inb.at[p], sem_i.at[p])

    def start_out(k, p):
        row = pl.multiple_of((wid + _NW * k) * (_SB // 2), _SB // 2)
        pltpu.async_copy(
            outb.at[p], out_hbm.at[pl.ds(row, _SB // 2)], sem_o.at[p])

    def wait_in(p):
        pltpu.make_async_copy(tt_hbm.at[:, pl.ds(0, _SB)], inb.at[p],
                              sem_i.at[p]).wait()

    def wait_out(p):
        pltpu.make_async_copy(outb.at[p], out_hbm.at[pl.ds(0, _SB // 2)],
                              sem_o.at[p]).wait()

    def transpose_block(p):
        @plsc.parallel_loop(0, _SB // 16, unroll=4)
        def body(s0):
            cols = s0 * 16 + iota16
            rows = s0 * 8 + (iota16 >> 1)
            half = (iota16 & 1) * D_MODEL
            for j in range(16):
                perm = (iota16 + j) & 15
                for c0 in range(4):
                    vals = plsc.load_gather(inb.at[p], [c0 * 16 + perm, cols])
                    plsc.store_scatter(outb.at[p],
                                       [rows, half + c0 * 16 + perm], vals)

    start_in(0, 0)

    def blk_body(k, carry):
        p = k & 1

        @pl.when(k + 1 < nblk)
        def _():
            start_in(k + 1, 1 - p)

        wait_in(p)

        @pl.when(k >= 2)
        def _():
            wait_out(p)

        transpose_block(p)
        start_out(k, p)
        return carry

    lax.fori_loop(0, nblk, blk_body, 0)
    wait_out((nblk - 1) & 1)
    wait_out((nblk - 2) & 1)

    @pl.when(wid == 0)
    def _():
        pltpu.sync_copy(tail_hbm, outb.at[0, pl.ds(0, _TAIL // 2)])
        pltpu.sync_copy(outb.at[0, pl.ds(0, _TAIL // 2)],
                        out_hbm.at[pl.ds(_NFULL * (_SB // 2), _TAIL // 2)])


@functools.partial(
    pl.kernel,
    mesh=_mesh,
    compiler_params=_params,
    out_type=jax.ShapeDtypeStruct((D_MODEL, BATCH), jnp.float32),
    scratch_types=[
        pltpu.VMEM((_B_PER_W,), jnp.int32),
        pltpu.VMEM((_B_PER_W,), jnp.int32),
        pltpu.VMEM((_B_PER_W,), jnp.int32),
        pltpu.VMEM((_B_PER_W, 2 * D_MODEL), jnp.float32),
        pltpu.VMEM((D_MODEL, _B_PER_W), jnp.float32),
        pltpu.SemaphoreType.DMA,
    ],
)
def _gather_kernel(idx_hbm, table_hbm, out_hbm, idx_v, pair_v, half_v,
                   rows_v, obuf, sem):
    wid = lax.axis_index("s") * _NC + lax.axis_index("c")
    base = wid * _B_PER_W
    iota16 = lax.broadcasted_iota(jnp.int32, (16,), 0)
    pltpu.sync_copy(idx_hbm.at[pl.ds(base, _B_PER_W)], idx_v)

    @plsc.parallel_loop(0, _B_PER_W // 16, unroll=4)
    def _prep(g):
        v = idx_v[pl.ds(g * 16, 16)]
        pair_v[pl.ds(g * 16, 16)] = v >> 1
        half_v[pl.ds(g * 16, 16)] = (v & 1) * D_MODEL

    copies = [
        pltpu.async_copy(
            table_hbm.at[pair_v.at[pl.ds(j * _CHUNK, _CHUNK)]],
            rows_v.at[pl.ds(j * _CHUNK, _CHUNK)],
            sem,
        )
        for j in range(_NCHUNK)
    ]
    for c in copies:
        c.wait()

    @plsc.parallel_loop(0, _B_PER_W // 16, unroll=4)
    def _tr(b0):
        halfvec = half_v[pl.ds(b0 * 16, 16)]
        brows = b0 * 16 + iota16
        for j in range(16):
            perm = (iota16 + j) & 15
            for c0 in range(4):
                vals = plsc.load_gather(
                    rows_v, [brows, halfvec + c0 * 16 + perm])
                plsc.store_scatter(obuf, [c0 * 16 + perm, brows], vals)

    pltpu.sync_copy(obuf, out_hbm.at[:, pl.ds(base, _B_PER_W)])


def kernel(species_ids, embedding):
    idx = species_ids.astype(jnp.int32)
    tt = embedding.T
    tail = embedding[_NFULL * _SB:].reshape(_TAIL // 2, 2 * D_MODEL)
    table2 = _relayout_kernel(tt, tail)
    out_cm = _gather_kernel(idx, table2)
    return out_cm.T[:, None, :]
